# baseline (speedup 1.0000x reference)
import jax
import jax.numpy as jnp
from jax import lax
from jax.experimental import pallas as pl
from jax.experimental.pallas import tpu as pltpu
from jax.experimental.pallas import tpu_sc as plsc

N = 3000
NP = 3072
T = 4
F = 128
H = 64
C = T * H
E = 48000
NREL = 6
NRW = 12
NS = 16
BAND = NP // NS
ACCR = 224
TRASH = BAND
CAP = 3456
NCH = CAP // 128
SCH = 96
SNCH = CAP // SCH


def _deg_body(idx_hbm, zeros_hbm, out_hbm, acc, idx_v):
    c = lax.axis_index("c")
    s = lax.axis_index("s")
    lane = lax.iota(jnp.int32, 16)
    ones16 = jnp.ones((16,), jnp.float32)

    def do_v(i, carry):
        v = 2 * i + c
        pltpu.sync_copy(zeros_hbm, acc)

        def chunk(j, c2):
            base = (v * NS + s) * CAP + j * 128
            pltpu.sync_copy(idx_hbm.at[pl.ds(base, 128)], idx_v)

            def group(g, c3):
                lrow = idx_v[pl.ds(g * 16, 16)]
                plsc.addupdate_scatter(acc, [lrow, lane], ones16)
                return c3

            lax.fori_loop(0, 8, group, 0)
            return c2

        lax.fori_loop(0, NCH, chunk, 0)
        pltpu.sync_copy(acc.at[pl.ds(0, BAND)],
                        out_hbm.at[pl.ds(v * NP + s * BAND, BAND)])
        return carry

    lax.fori_loop(0, NRW // 2, do_v, 0)


def _seg_body(z_hbm, src_hbm, dst_hbm, zeros_hbm, out_hbm,
              acc, srci, dsti, slab_a, slab_b, sem_a, sem_b):
    c = lax.axis_index("c")
    s = lax.axis_index("s")
    lane = lax.iota(jnp.int32, 16)
    cols = [lane + p * 16 for p in range(16)]

    def gstart(j, slab, sem):
        pltpu.async_copy(z_hbm.at[srci.at[pl.ds(j * SCH, SCH)]], slab, sem)

    def gwait(j, slab, sem):
        pltpu.make_async_copy(
            z_hbm.at[srci.at[pl.ds(j * SCH, SCH)]], slab, sem).wait()

    def process(j, slab):
        def group(g, c3):
            lrow = dsti[pl.ds(j * SCH + g * 16, 16)]
            for eb in range(2):
                bcs, gvs = [], []
                for e in range(eb * 8, eb * 8 + 8):
                    esel = jnp.full((16,), e, jnp.int32)
                    bcs.append(lrow.at[esel].get(mode="promise_in_bounds"))
                    gvs.append(jnp.full((16,), g * 16 + e, jnp.int32))
                for p in range(16):
                    for ei in range(8):
                        val = plsc.load_gather(slab, [gvs[ei], cols[p]])
                        plsc.addupdate_scatter(acc, [bcs[ei], cols[p]], val)
            return c3

        lax.fori_loop(0, SCH // 16, group, 0)

    def do_rw(i, carry):
        rw = 2 * i + c
        rel = lax.rem(rw, NREL)
        pltpu.sync_copy(zeros_hbm, acc)
        pltpu.sync_copy(src_hbm.at[pl.ds((rw * NS + s) * CAP, CAP)], srci)
        pltpu.sync_copy(dst_hbm.at[pl.ds((rel * NS + s) * CAP, CAP)], dsti)
        gstart(0, slab_a, sem_a)

        def pair(k, c2):
            j0 = 2 * k
            gstart(j0 + 1, slab_b, sem_b)
            gwait(j0, slab_a, sem_a)
            process(j0, slab_a)

            @pl.when(j0 + 2 < SNCH)
            def _issue():
                gstart(j0 + 2, slab_a, sem_a)

            gwait(j0 + 1, slab_b, sem_b)
            process(j0 + 1, slab_b)
            return c2

        lax.fori_loop(0, SNCH // 2, pair, 0)
        pltpu.sync_copy(acc.at[pl.ds(0, BAND)],
                        out_hbm.at[pl.ds(rw * NP + s * BAND, BAND)])
        return carry

    lax.fori_loop(0, NRW // 2, do_rw, 0)


def _a_body(x_ref, w_ref, deg_ref, out_ref):
    x = x_ref[0]
    w = w_ref[0]
    zs = [jnp.dot(x[:, t * F:(t + 1) * F], w,
                  preferred_element_type=jnp.float32) for t in range(T)]
    z = jnp.concatenate(zs, axis=1)
    d = deg_ref[0, :, 0:1]
    out_ref[0] = z * lax.rsqrt(jnp.maximum(d, 1.0))


def _b1_body(a_ref, deg_ref, bsec_ref, wgm_ref, outg_ref, outl_ref):
    sall = lax.rsqrt(jnp.maximum(deg_ref[...], 1.0))
    outs = [a_ref[0, r] * sall[:, r:r + 1] for r in range(NREL)]
    g0 = jax.nn.relu(outs[1] + bsec_ref[0:1, :])
    g1 = jax.nn.relu((outs[0] + outs[2] + outs[5]) * (1.0 / 3.0)
                     + bsec_ref[1:2, :])
    g2 = jax.nn.relu((outs[3] + outs[4]) * 0.5 + bsec_ref[2:3, :])
    zero = jnp.zeros_like(g0[:, 0:1])
    for i, g in enumerate((g0, g1, g2)):
        outg_ref[0, i] = g
        lcols = [jnp.sum(g * wgm_ref[t:t + 1, :], axis=1, keepdims=True)
                 for t in range(T)]
        outl_ref[0, i] = jnp.concatenate(lcols + [zero] * 4, axis=1)


def _b2_body(g_ref, l_ref, tm_ref, fold_ref, out_ref):
    valid = lax.broadcasted_iota(jnp.int32, (NP, 1), 0) < N
    acc = jnp.zeros((1, C), jnp.float32)
    for t in range(T):
        ls = [jnp.where(valid, l_ref[0, i, :, t:t + 1], -1e30)
              for i in range(3)]
        m = jnp.maximum(jnp.maximum(jnp.max(ls[0]), jnp.max(ls[1])),
                        jnp.max(ls[2]))
        ps = [jnp.exp(l - m) for l in ls]
        ssum = jnp.sum(ps[0]) + jnp.sum(ps[1]) + jnp.sum(ps[2])
        num = sum(jnp.sum(p * g_ref[0, i], axis=0, keepdims=True)
                  for i, p in enumerate(ps))
        acc = acc + (num / ssum) * tm_ref[t:t + 1, :]
    w = pl.program_id(0)
    out_ref[pl.ds(w, 1), :] = jnp.dot(acc, fold_ref[...],
                                      preferred_element_type=jnp.float32)


def _c_body(x_ref, wi0, bi0, wf0, bf0, wg0, bg0, wo0, bo0,
            wi1, bi1, wf1, bf1, wg1, bg1, wo1, bo1, wlin_ref, out_ref):
    def cell(x, wi, bi, wf, bf, wg, bg, wo, bo):
        ii = jax.nn.sigmoid(jnp.dot(x, wi[...],
                                    preferred_element_type=jnp.float32)
                            + bi[...])
        gg = jnp.tanh(jnp.dot(x, wg[...],
                              preferred_element_type=jnp.float32) + bg[...])
        oo = jax.nn.sigmoid(jnp.dot(x, wo[...],
                                    preferred_element_type=jnp.float32)
                            + bo[...])
        cc = ii * gg
        return oo * jnp.tanh(cc)

    x = x_ref[...]
    h = cell(x, wi0, bi0, wf0, bf0, wg0, bg0, wo0, bo0)
    h = cell(h, wi1, bi1, wf1, bf1, wg1, bg1, wo1, bo1)
    h = jax.nn.relu(h)
    logit = jnp.sum(h * wlin_ref[...], axis=1, keepdims=True)
    m = jnp.max(logit)
    e = jnp.exp(logit - m)
    out_ref[...] = e / jnp.sum(e)


def _bucketize(vals, others):
    order = jnp.argsort(vals, axis=1, stable=True)
    sortd = jnp.take_along_axis(vals, order, axis=1)
    bounds = jnp.arange(NS + 1, dtype=jnp.int32) * BAND
    starts = jax.vmap(lambda row: jnp.searchsorted(row, bounds))(sortd)
    pos = starts[:, :NS, None] + jnp.arange(CAP, dtype=jnp.int32)[None, None]
    validb = pos < starts[:, 1:, None]
    posc = jnp.minimum(pos, E - 1).reshape(NREL, -1)
    eid = jnp.take_along_axis(order, posc, axis=1)
    v_taken = jnp.take_along_axis(vals, eid, axis=1).reshape(NREL, NS, CAP)
    lrow = jnp.where(validb, v_taken - bounds[:NS][None, :, None], TRASH)
    o_taken = jnp.take_along_axis(others, eid, axis=1).reshape(NREL, NS, CAP)
    o_taken = jnp.where(validb, o_taken, N)
    return lrow.astype(jnp.int32), o_taken.astype(jnp.int32)


def kernel(node_feat_w0, inst_feat_w0, svc_feat_w0, node_feat_w1,
           inst_feat_w1, svc_feat_w1, ei_svc_call, ei_inst_node,
           ei_node_inst, ei_inst_inst, ei_svc_inst, ei_inst_svc,
           W_svc_call, b_svc_call, W_inst_node, b_inst_node, W_node_inst,
           b_node_inst, W_inst_inst, b_inst_inst, W_svc_inst, b_svc_inst,
           W_inst_svc, b_inst_svc,
           lstm1_Wih0, lstm1_Whh0, lstm1_bih0, lstm1_bhh0,
           lstm1_Wih1, lstm1_Whh1, lstm1_bih1, lstm1_bhh1,
           lstm2_Wih0, lstm2_Whh0, lstm2_bih0, lstm2_bhh0,
           lstm2_Wih1, lstm2_Whh1, lstm2_bih1, lstm2_bhh1,
           Wg, bg, Wlin, blin):
    f32 = jnp.float32
    eis = [ei_node_inst, ei_inst_node, ei_inst_inst, ei_inst_svc,
           ei_svc_call, ei_svc_inst]
    Ws = [W_node_inst, W_inst_node, W_inst_inst, W_inst_svc,
          W_svc_call, W_svc_inst]
    bs = [b_node_inst, b_inst_node, b_inst_inst, b_inst_svc,
          b_svc_call, b_svc_inst]

    src6 = jnp.stack([ei[0] for ei in eis]).astype(jnp.int32)
    dst6 = jnp.stack([ei[1] for ei in eis]).astype(jnp.int32)

    dstl, srcg = _bucketize(dst6, src6)
    srcl, _ = _bucketize(src6, dst6)

    deg_idx = jnp.concatenate([srcl.reshape(-1), dstl.reshape(-1)])
    rwoff = (jnp.arange(NRW, dtype=jnp.int32) * NP)[:, None, None]
    src12 = (jnp.tile(srcg, (2, 1, 1)) + rwoff).reshape(-1)
    dstl_f = dstl.reshape(-1)

    mesh = plsc.VectorSubcoreMesh(core_axis_name="c", subcore_axis_name="s",
                                  num_cores=2, num_subcores=NS)

    deg_flat = pl.kernel(
        _deg_body,
        out_type=jax.ShapeDtypeStruct((NRW * NP, 16), f32),
        mesh=mesh,
        compiler_params=pltpu.CompilerParams(needs_layout_passes=False),
        scratch_types=[
            pltpu.VMEM((ACCR, 16), f32),
            pltpu.VMEM((128,), jnp.int32),
        ],
    )(deg_idx, jnp.zeros((ACCR, 16), f32))
    deg = deg_flat.sum(axis=1).reshape(2, NREL, NP)
    deg_src, deg_dst = deg[0], deg[1]

    x_all = jnp.stack([node_feat_w0, inst_feat_w0, svc_feat_w0,
                       node_feat_w1, inst_feat_w1, svc_feat_w1])
    x_all = jnp.pad(x_all, ((0, 0), (0, NP - N), (0, 0), (0, 0)))
    x_all = x_all.reshape(NREL, NP, T * F)
    W12 = jnp.tile(jnp.stack(Ws), (2, 1, 1))
    degsrc12 = jnp.broadcast_to(
        jnp.tile(deg_src, (2, 1))[:, :, None], (NRW, NP, 8))

    BM = 768
    NBM = NP // BM

    def xmap(r, m):
        rel = lax.rem(r, NREL)
        w = r // NREL
        ty = (rel >= 1).astype(jnp.int32) + (rel >= 4).astype(jnp.int32)
        return (ty + 3 * w, m, 0)

    z_all = pl.pallas_call(
        _a_body,
        grid=(NRW, NBM),
        in_specs=[
            pl.BlockSpec((1, BM, T * F), xmap),
            pl.BlockSpec((1, F, H), lambda r, m: (r, 0, 0)),
            pl.BlockSpec((1, BM, 8), lambda r, m: (r, m, 0)),
        ],
        out_specs=pl.BlockSpec((1, BM, C), lambda r, m: (r, m, 0)),
        out_shape=jax.ShapeDtypeStruct((NRW, NP, C), f32),
    )(x_all, W12, degsrc12)

    agg_flat = pl.kernel(
        _seg_body,
        out_type=jax.ShapeDtypeStruct((NRW * NP, C), f32),
        mesh=mesh,
        compiler_params=pltpu.CompilerParams(needs_layout_passes=False),
        scratch_types=[
            pltpu.VMEM((ACCR, C), f32),
            pltpu.VMEM((CAP,), jnp.int32),
            pltpu.VMEM((CAP,), jnp.int32),
            pltpu.VMEM((SCH, C), f32),
            pltpu.VMEM((SCH, C), f32),
            pltpu.SemaphoreType.DMA,
            pltpu.SemaphoreType.DMA,
        ],
    )(z_all.reshape(NRW * NP, C), src12, dstl_f,
      jnp.zeros((ACCR, C), f32))
    agg4 = agg_flat.reshape(2, NREL, NP, C)

    degdstT = jnp.concatenate(
        [deg_dst.T, jnp.ones((NP, 2), f32)], axis=1)
    b_t = [jnp.tile(b, (T,)) for b in bs]
    bsec = jnp.stack([b_t[1],
                      (b_t[0] + b_t[2] + b_t[5]) / 3.0,
                      (b_t[3] + b_t[4]) / 2.0,
                      jnp.zeros((C,), f32), jnp.zeros((C,), f32),
                      jnp.zeros((C,), f32), jnp.zeros((C,), f32),
                      jnp.zeros((C,), f32)])
    tmask4 = ((jnp.arange(C) // H)[None, :]
              == jnp.arange(T)[:, None]).astype(f32)
    wg_t = jnp.tile(Wg[0], (T,))[None, :]
    wgm = jnp.concatenate([wg_t * tmask4, jnp.zeros((4, C), f32)])
    tm8 = jnp.concatenate([tmask4, jnp.zeros((4, C), f32)])
    fold = jnp.tile(jnp.eye(H, dtype=f32), (T, 1))

    gts, logits = pl.pallas_call(
        _b1_body,
        grid=(2, NBM),
        in_specs=[
            pl.BlockSpec((1, NREL, BM, C), lambda w, m: (w, 0, m, 0)),
            pl.BlockSpec((BM, 8), lambda w, m: (m, 0)),
            pl.BlockSpec((8, C), lambda w, m: (0, 0)),
            pl.BlockSpec((8, C), lambda w, m: (0, 0)),
        ],
        out_specs=[
            pl.BlockSpec((1, 3, BM, C), lambda w, m: (w, 0, m, 0)),
            pl.BlockSpec((1, 3, BM, 8), lambda w, m: (w, 0, m, 0)),
        ],
        out_shape=[
            jax.ShapeDtypeStruct((2, 3, NP, C), f32),
            jax.ShapeDtypeStruct((2, 3, NP, 8), f32),
        ],
    )(agg4, degdstT, bsec, wgm)

    r_all = pl.pallas_call(
        _b2_body,
        grid=(2,),
        in_specs=[
            pl.BlockSpec((1, 3, NP, C), lambda w: (w, 0, 0, 0)),
            pl.BlockSpec((1, 3, NP, 8), lambda w: (w, 0, 0, 0)),
            pl.BlockSpec((8, C), lambda w: (0, 0)),
            pl.BlockSpec((C, H), lambda w: (0, 0)),
        ],
        out_specs=pl.BlockSpec((2, H), lambda w: (0, 0)),
        out_shape=jax.ShapeDtypeStruct((2, H), f32),
    )(gts, logits, tm8, fold)

    def gates(Wih, bih, bhh):
        bsum = bih + bhh
        out = []
        for k in range(4):
            out.append(Wih[k * H:(k + 1) * H, :].T)
            out.append(bsum[None, k * H:(k + 1) * H])
        return out

    args = ([r_all] + gates(lstm2_Wih0, lstm2_bih0, lstm2_bhh0)
            + gates(lstm2_Wih1, lstm2_bih1, lstm2_bhh1) + [Wlin])
    out = pl.pallas_call(
        _c_body,
        out_shape=jax.ShapeDtypeStruct((2, 1), f32),
    )(*args)
    return out.reshape(2, 1, 1)


# speedup vs baseline: 1.0274x; 1.0274x over previous
import jax
import jax.numpy as jnp
from jax import lax
from jax.experimental import pallas as pl
from jax.experimental.pallas import tpu as pltpu
from jax.experimental.pallas import tpu_sc as plsc

N = 3000
NP = 3072
T = 4
F = 128
H = 64
C = T * H
E = 48000
NREL = 6
NRW = 12
NS = 16
BAND = NP // NS
ACCR = 224
TRASH = BAND
CAP = 3456
NCH = CAP // 128
SCH = 96
SNCH = CAP // SCH


def _deg_body(idx_hbm, zeros_hbm, out_hbm, acc, idx_v):
    c = lax.axis_index("c")
    s = lax.axis_index("s")
    lane = lax.iota(jnp.int32, 16)
    ones16 = jnp.ones((16,), jnp.float32)

    def do_v(i, carry):
        v = 2 * i + c
        pltpu.sync_copy(zeros_hbm, acc)

        def chunk(j, c2):
            base = (v * NS + s) * CAP + j * 128
            pltpu.sync_copy(idx_hbm.at[pl.ds(base, 128)], idx_v)

            def group(g, c3):
                lrow = idx_v[pl.ds(g * 16, 16)]
                plsc.addupdate_scatter(acc, [lrow, lane], ones16)
                return c3

            lax.fori_loop(0, 8, group, 0)
            return c2

        lax.fori_loop(0, NCH, chunk, 0)
        pltpu.sync_copy(acc.at[pl.ds(0, BAND)],
                        out_hbm.at[pl.ds(v * NP + s * BAND, BAND)])
        return carry

    lax.fori_loop(0, NRW // 2, do_v, 0)


def _seg_body(z_hbm, src_hbm, dst_hbm, zeros_hbm, out_hbm,
              acc, srci, dsti, slab_a, slab_b, sem_a, sem_b):
    c = lax.axis_index("c")
    s = lax.axis_index("s")
    lane = lax.iota(jnp.int32, 16)
    cols = [lane + p * 16 for p in range(16)]

    def gstart(j, slab, sem):
        pltpu.async_copy(z_hbm.at[srci.at[pl.ds(j * SCH, SCH)]], slab, sem)

    def gwait(j, slab, sem):
        pltpu.make_async_copy(
            z_hbm.at[srci.at[pl.ds(j * SCH, SCH)]], slab, sem).wait()

    def process(j, slab):
        def group(g, c3):
            lrow = dsti[pl.ds(j * SCH + g * 16, 16)]
            for eb in range(2):
                bcs, gvs = [], []
                for e in range(eb * 8, eb * 8 + 8):
                    esel = jnp.full((16,), e, jnp.int32)
                    bcs.append(lrow.at[esel].get(mode="promise_in_bounds"))
                    gvs.append(jnp.full((16,), g * 16 + e, jnp.int32))
                for p in range(16):
                    for ei in range(8):
                        val = plsc.load_gather(slab, [gvs[ei], cols[p]])
                        plsc.addupdate_scatter(acc, [bcs[ei], cols[p]], val)
            return c3

        lax.fori_loop(0, SCH // 16, group, 0)

    def do_rw(i, carry):
        rw = 2 * i + c
        rel = lax.rem(rw, NREL)
        pltpu.sync_copy(zeros_hbm, acc)
        pltpu.sync_copy(src_hbm.at[pl.ds((rw * NS + s) * CAP, CAP)], srci)
        pltpu.sync_copy(dst_hbm.at[pl.ds((rel * NS + s) * CAP, CAP)], dsti)
        gstart(0, slab_a, sem_a)

        def pair(k, c2):
            j0 = 2 * k
            gstart(j0 + 1, slab_b, sem_b)
            gwait(j0, slab_a, sem_a)
            process(j0, slab_a)

            @pl.when(j0 + 2 < SNCH)
            def _issue():
                gstart(j0 + 2, slab_a, sem_a)

            gwait(j0 + 1, slab_b, sem_b)
            process(j0 + 1, slab_b)
            return c2

        lax.fori_loop(0, SNCH // 2, pair, 0)
        pltpu.sync_copy(acc.at[pl.ds(0, BAND)],
                        out_hbm.at[pl.ds(rw * NP + s * BAND, BAND)])
        return carry

    lax.fori_loop(0, NRW // 2, do_rw, 0)


def _a_body(x_ref, w_ref, deg_ref, out_ref):
    x = x_ref[0]
    w = w_ref[0]
    zs = [jnp.dot(x[:, t * F:(t + 1) * F], w,
                  preferred_element_type=jnp.float32) for t in range(T)]
    z = jnp.concatenate(zs, axis=1)
    d = deg_ref[0, :, 0:1]
    out_ref[0] = z * lax.rsqrt(jnp.maximum(d, 1.0))


def _b1_body(a_ref, deg_ref, bsec_ref, wgm_ref, outg_ref, outl_ref):
    sall = lax.rsqrt(jnp.maximum(deg_ref[...], 1.0))
    outs = [a_ref[0, r] * sall[:, r:r + 1] for r in range(NREL)]
    g0 = jax.nn.relu(outs[1] + bsec_ref[0:1, :])
    g1 = jax.nn.relu((outs[0] + outs[2] + outs[5]) * (1.0 / 3.0)
                     + bsec_ref[1:2, :])
    g2 = jax.nn.relu((outs[3] + outs[4]) * 0.5 + bsec_ref[2:3, :])
    zero = jnp.zeros_like(g0[:, 0:1])
    for i, g in enumerate((g0, g1, g2)):
        outg_ref[0, i] = g
        lcols = [jnp.sum(g * wgm_ref[t:t + 1, :], axis=1, keepdims=True)
                 for t in range(T)]
        outl_ref[0, i] = jnp.concatenate(lcols + [zero] * 4, axis=1)


def _b2_body(g_ref, l_ref, tm_ref, fold_ref, out_ref):
    valid = lax.broadcasted_iota(jnp.int32, (NP, 1), 0) < N
    acc = jnp.zeros((1, C), jnp.float32)
    for t in range(T):
        ls = [jnp.where(valid, l_ref[0, i, :, t:t + 1], -1e30)
              for i in range(3)]
        m = jnp.maximum(jnp.maximum(jnp.max(ls[0]), jnp.max(ls[1])),
                        jnp.max(ls[2]))
        ps = [jnp.exp(l - m) for l in ls]
        ssum = jnp.sum(ps[0]) + jnp.sum(ps[1]) + jnp.sum(ps[2])
        num = sum(jnp.sum(p * g_ref[0, i], axis=0, keepdims=True)
                  for i, p in enumerate(ps))
        acc = acc + (num / ssum) * tm_ref[t:t + 1, :]
    w = pl.program_id(0)
    out_ref[pl.ds(w, 1), :] = jnp.dot(acc, fold_ref[...],
                                      preferred_element_type=jnp.float32)


def _c_body(x_ref, wi0, bi0, wf0, bf0, wg0, bg0, wo0, bo0,
            wi1, bi1, wf1, bf1, wg1, bg1, wo1, bo1, wlin_ref, out_ref):
    def cell(x, wi, bi, wf, bf, wg, bg, wo, bo):
        ii = jax.nn.sigmoid(jnp.dot(x, wi[...],
                                    preferred_element_type=jnp.float32)
                            + bi[...])
        gg = jnp.tanh(jnp.dot(x, wg[...],
                              preferred_element_type=jnp.float32) + bg[...])
        oo = jax.nn.sigmoid(jnp.dot(x, wo[...],
                                    preferred_element_type=jnp.float32)
                            + bo[...])
        cc = ii * gg
        return oo * jnp.tanh(cc)

    x = x_ref[...]
    h = cell(x, wi0, bi0, wf0, bf0, wg0, bg0, wo0, bo0)
    h = cell(h, wi1, bi1, wf1, bf1, wg1, bg1, wo1, bo1)
    h = jax.nn.relu(h)
    logit = jnp.sum(h * wlin_ref[...], axis=1, keepdims=True)
    m = jnp.max(logit)
    e = jnp.exp(logit - m)
    out_ref[...] = e / jnp.sum(e)


def _bucketize(vals, others):
    key = ((vals.astype(jnp.uint32) << 16)
           | jnp.arange(E, dtype=jnp.uint32)[None, :])
    keys = jnp.sort(key, axis=1)
    order = (keys & 0xFFFF).astype(jnp.int32)
    sortd = (keys >> 16).astype(jnp.int32)
    bounds = jnp.arange(NS + 1, dtype=jnp.int32) * BAND
    starts = jax.vmap(lambda row: jnp.searchsorted(row, bounds))(sortd)
    pos = starts[:, :NS, None] + jnp.arange(CAP, dtype=jnp.int32)[None, None]
    validb = pos < starts[:, 1:, None]
    posc = jnp.minimum(pos, E - 1).reshape(NREL, -1)
    eid = jnp.take_along_axis(order, posc, axis=1)
    v_taken = jnp.take_along_axis(sortd, posc, axis=1).reshape(NREL, NS, CAP)
    lrow = jnp.where(validb, v_taken - bounds[:NS][None, :, None], TRASH)
    o_taken = jnp.take_along_axis(others, eid, axis=1).reshape(NREL, NS, CAP)
    o_taken = jnp.where(validb, o_taken, N)
    return lrow.astype(jnp.int32), o_taken.astype(jnp.int32)


def kernel(node_feat_w0, inst_feat_w0, svc_feat_w0, node_feat_w1,
           inst_feat_w1, svc_feat_w1, ei_svc_call, ei_inst_node,
           ei_node_inst, ei_inst_inst, ei_svc_inst, ei_inst_svc,
           W_svc_call, b_svc_call, W_inst_node, b_inst_node, W_node_inst,
           b_node_inst, W_inst_inst, b_inst_inst, W_svc_inst, b_svc_inst,
           W_inst_svc, b_inst_svc,
           lstm1_Wih0, lstm1_Whh0, lstm1_bih0, lstm1_bhh0,
           lstm1_Wih1, lstm1_Whh1, lstm1_bih1, lstm1_bhh1,
           lstm2_Wih0, lstm2_Whh0, lstm2_bih0, lstm2_bhh0,
           lstm2_Wih1, lstm2_Whh1, lstm2_bih1, lstm2_bhh1,
           Wg, bg, Wlin, blin):
    f32 = jnp.float32
    eis = [ei_node_inst, ei_inst_node, ei_inst_inst, ei_inst_svc,
           ei_svc_call, ei_svc_inst]
    Ws = [W_node_inst, W_inst_node, W_inst_inst, W_inst_svc,
          W_svc_call, W_svc_inst]
    bs = [b_node_inst, b_inst_node, b_inst_inst, b_inst_svc,
          b_svc_call, b_svc_inst]

    src6 = jnp.stack([ei[0] for ei in eis]).astype(jnp.int32)
    dst6 = jnp.stack([ei[1] for ei in eis]).astype(jnp.int32)

    dstl, srcg = _bucketize(dst6, src6)
    srcl, _ = _bucketize(src6, dst6)

    deg_idx = jnp.concatenate([srcl.reshape(-1), dstl.reshape(-1)])
    rwoff = (jnp.arange(NRW, dtype=jnp.int32) * NP)[:, None, None]
    src12 = (jnp.tile(srcg, (2, 1, 1)) + rwoff).reshape(-1)
    dstl_f = dstl.reshape(-1)

    mesh = plsc.VectorSubcoreMesh(core_axis_name="c", subcore_axis_name="s",
                                  num_cores=2, num_subcores=NS)

    deg_flat = pl.kernel(
        _deg_body,
        out_type=jax.ShapeDtypeStruct((NRW * NP, 16), f32),
        mesh=mesh,
        compiler_params=pltpu.CompilerParams(needs_layout_passes=False),
        scratch_types=[
            pltpu.VMEM((ACCR, 16), f32),
            pltpu.VMEM((128,), jnp.int32),
        ],
    )(deg_idx, jnp.zeros((ACCR, 16), f32))
    deg = deg_flat.sum(axis=1).reshape(2, NREL, NP)
    deg_src, deg_dst = deg[0], deg[1]

    x_all = jnp.stack([node_feat_w0, inst_feat_w0, svc_feat_w0,
                       node_feat_w1, inst_feat_w1, svc_feat_w1])
    x_all = jnp.pad(x_all, ((0, 0), (0, NP - N), (0, 0), (0, 0)))
    x_all = x_all.reshape(NREL, NP, T * F)
    W12 = jnp.tile(jnp.stack(Ws), (2, 1, 1))
    degsrc12 = jnp.broadcast_to(
        jnp.tile(deg_src, (2, 1))[:, :, None], (NRW, NP, 8))

    BM = 768
    NBM = NP // BM

    def xmap(r, m):
        rel = lax.rem(r, NREL)
        w = r // NREL
        ty = (rel >= 1).astype(jnp.int32) + (rel >= 4).astype(jnp.int32)
        return (ty + 3 * w, m, 0)

    z_all = pl.pallas_call(
        _a_body,
        grid=(NRW, NBM),
        in_specs=[
            pl.BlockSpec((1, BM, T * F), xmap),
            pl.BlockSpec((1, F, H), lambda r, m: (r, 0, 0)),
            pl.BlockSpec((1, BM, 8), lambda r, m: (r, m, 0)),
        ],
        out_specs=pl.BlockSpec((1, BM, C), lambda r, m: (r, m, 0)),
        out_shape=jax.ShapeDtypeStruct((NRW, NP, C), f32),
    )(x_all, W12, degsrc12)

    agg_flat = pl.kernel(
        _seg_body,
        out_type=jax.ShapeDtypeStruct((NRW * NP, C), f32),
        mesh=mesh,
        compiler_params=pltpu.CompilerParams(needs_layout_passes=False),
        scratch_types=[
            pltpu.VMEM((ACCR, C), f32),
            pltpu.VMEM((CAP,), jnp.int32),
            pltpu.VMEM((CAP,), jnp.int32),
            pltpu.VMEM((SCH, C), f32),
            pltpu.VMEM((SCH, C), f32),
            pltpu.SemaphoreType.DMA,
            pltpu.SemaphoreType.DMA,
        ],
    )(z_all.reshape(NRW * NP, C), src12, dstl_f,
      jnp.zeros((ACCR, C), f32))
    agg4 = agg_flat.reshape(2, NREL, NP, C)

    degdstT = jnp.concatenate(
        [deg_dst.T, jnp.ones((NP, 2), f32)], axis=1)
    b_t = [jnp.tile(b, (T,)) for b in bs]
    bsec = jnp.stack([b_t[1],
                      (b_t[0] + b_t[2] + b_t[5]) / 3.0,
                      (b_t[3] + b_t[4]) / 2.0,
                      jnp.zeros((C,), f32), jnp.zeros((C,), f32),
                      jnp.zeros((C,), f32), jnp.zeros((C,), f32),
                      jnp.zeros((C,), f32)])
    tmask4 = ((jnp.arange(C) // H)[None, :]
              == jnp.arange(T)[:, None]).astype(f32)
    wg_t = jnp.tile(Wg[0], (T,))[None, :]
    wgm = jnp.concatenate([wg_t * tmask4, jnp.zeros((4, C), f32)])
    tm8 = jnp.concatenate([tmask4, jnp.zeros((4, C), f32)])
    fold = jnp.tile(jnp.eye(H, dtype=f32), (T, 1))

    gts, logits = pl.pallas_call(
        _b1_body,
        grid=(2, NBM),
        in_specs=[
            pl.BlockSpec((1, NREL, BM, C), lambda w, m: (w, 0, m, 0)),
            pl.BlockSpec((BM, 8), lambda w, m: (m, 0)),
            pl.BlockSpec((8, C), lambda w, m: (0, 0)),
            pl.BlockSpec((8, C), lambda w, m: (0, 0)),
        ],
        out_specs=[
            pl.BlockSpec((1, 3, BM, C), lambda w, m: (w, 0, m, 0)),
            pl.BlockSpec((1, 3, BM, 8), lambda w, m: (w, 0, m, 0)),
        ],
        out_shape=[
            jax.ShapeDtypeStruct((2, 3, NP, C), f32),
            jax.ShapeDtypeStruct((2, 3, NP, 8), f32),
        ],
    )(agg4, degdstT, bsec, wgm)

    r_all = pl.pallas_call(
        _b2_body,
        grid=(2,),
        in_specs=[
            pl.BlockSpec((1, 3, NP, C), lambda w: (w, 0, 0, 0)),
            pl.BlockSpec((1, 3, NP, 8), lambda w: (w, 0, 0, 0)),
            pl.BlockSpec((8, C), lambda w: (0, 0)),
            pl.BlockSpec((C, H), lambda w: (0, 0)),
        ],
        out_specs=pl.BlockSpec((2, H), lambda w: (0, 0)),
        out_shape=jax.ShapeDtypeStruct((2, H), f32),
    )(gts, logits, tm8, fold)

    def gates(Wih, bih, bhh):
        bsum = bih + bhh
        out = []
        for k in range(4):
            out.append(Wih[k * H:(k + 1) * H, :].T)
            out.append(bsum[None, k * H:(k + 1) * H])
        return out

    args = ([r_all] + gates(lstm2_Wih0, lstm2_bih0, lstm2_bhh0)
            + gates(lstm2_Wih1, lstm2_bih1, lstm2_bhh1) + [Wlin])
    out = pl.pallas_call(
        _c_body,
        out_shape=jax.ShapeDtypeStruct((2, 1), f32),
    )(*args)
    return out.reshape(2, 1, 1)
